# baseline (speedup 1.0000x reference)
import jax
import jax.numpy as jnp
from jax import lax
from jax.experimental import pallas as pl
from jax.experimental.pallas import tpu as pltpu
from jax.experimental.pallas import tpu_sc as plsc

B = 4096
NNUM = 13
NCAT = 26
V = 100000
D = 64

NC = 2
NS = 16
NW = NC * NS
CHUNK = B // NW


def _num_body(num_ref, w_ref, b_ref, out_ref):
    out_ref[...] = (num_ref[...][:, :, None] * w_ref[...][None, :, :]
                    + b_ref[...][None, :, :])


def _num_embed(numerical, num_W, num_bias):
    bb = 1024
    return pl.pallas_call(
        _num_body,
        grid=(B // bb,),
        in_specs=[
            pl.BlockSpec((bb, NNUM), lambda i: (i, 0)),
            pl.BlockSpec((NNUM, D), lambda i: (0, 0)),
            pl.BlockSpec((NNUM, D), lambda i: (0, 0)),
        ],
        out_specs=pl.BlockSpec((bb, NNUM, D), lambda i: (i, 0, 0)),
        out_shape=jax.ShapeDtypeStruct((B, NNUM, D), jnp.float32),
    )(numerical, num_W, num_bias)


def _sc_body(gidx_hbm, tab_hbm, bias_hbm, out_hbm, idx_v, rows_v, bias_v, gsem):
    wid = lax.axis_index("s") * NC + lax.axis_index("c")
    b0 = wid * CHUNK
    pltpu.sync_copy(bias_hbm, bias_v)

    def field_step(f, carry):
        pltpu.sync_copy(gidx_hbm.at[f, pl.ds(b0, CHUNK)], idx_v)
        pltpu.async_copy(tab_hbm.at[idx_v], rows_v, gsem).wait()
        bias_regs = [bias_v[f, pl.ds(16 * k, 16)] for k in range(D // 16)]

        def row_body(i, c):
            for k in range(D // 16):
                plsc.addupdate(rows_v.at[i, pl.ds(16 * k, 16)], bias_regs[k])
            return c

        lax.fori_loop(0, CHUNK, row_body, 0)
        pltpu.sync_copy(rows_v, out_hbm.at[pl.ds(b0, CHUNK), f])
        return carry

    lax.fori_loop(0, NCAT, field_step, 0)


def _cat_embed(gidx, tables_flat, cat_bias):
    mesh = plsc.VectorSubcoreMesh(core_axis_name="c", subcore_axis_name="s",
                                  num_cores=NC, num_subcores=NS)
    run = pl.kernel(
        _sc_body,
        out_type=jax.ShapeDtypeStruct((B, NCAT, D), jnp.float32),
        mesh=mesh,
        scratch_types=[
            pltpu.VMEM((CHUNK,), jnp.int32),
            pltpu.VMEM((CHUNK, D), jnp.float32),
            pltpu.VMEM((NCAT, D), jnp.float32),
            pltpu.SemaphoreType.DMA,
        ],
        compiler_params=pltpu.CompilerParams(use_tc_tiling_on_sc=False),
    )
    return run(gidx, tables_flat, cat_bias)


@jax.jit
def kernel(numerical, categorical, num_W, num_b, tables, num_token, cat_token,
           pos_enc):
    tables_flat = tables.reshape(NCAT * V, D)
    gidx = categorical.T + (jnp.arange(NCAT, dtype=jnp.int32) * V)[:, None]
    cat_bias = cat_token + pos_enc[NNUM:]
    num_bias = num_b + num_token + pos_enc[:NNUM]
    num_e = _num_embed(numerical, num_W, num_bias)
    cat_e = _cat_embed(gidx, tables_flat, cat_bias)
    catT = categorical.T
    keys = jnp.sort(catT * 4096 + jnp.arange(B, dtype=jnp.int32)[None, :],
                    axis=1)
    bounds = (jnp.arange(392, dtype=jnp.int32) << 20)
    offs = jnp.sum(keys[:, None, :] < bounds[None, :, None], axis=-1,
                   dtype=jnp.int32)
    probe = (jnp.min(keys) + jnp.min(offs)).astype(jnp.float32) * 1e-30
    out = jnp.concatenate([num_e, cat_e], axis=1)
    return out.at[0, 0, 0].add(probe)


# speedup vs baseline: 1.6990x; 1.6990x over previous
import jax
import jax.numpy as jnp
from jax import lax
from jax.experimental import pallas as pl
from jax.experimental.pallas import tpu as pltpu
from jax.experimental.pallas import tpu_sc as plsc

B = 4096
NNUM = 13
NCAT = 26
V = 100000
D = 64

NC = 2
NS = 16
NW = NC * NS

WIN = 256
NWINF = 391
WTOT = NCAT * NWINF
WPW = (WTOT + NW - 1) // NW
TAIL0 = 390 * WIN
TAILN = V - TAIL0
OUTROWS = B * NCAT + 8


def _num_body(num_ref, w_ref, b_ref, out_ref):
    out_ref[...] = (num_ref[...][:, :, None] * w_ref[...][None, :, :]
                    + b_ref[...][None, :, :])


def _num_embed(numerical, num_W, num_bias):
    bb = 1024
    return pl.pallas_call(
        _num_body,
        grid=(B // bb,),
        in_specs=[
            pl.BlockSpec((bb, NNUM), lambda i: (i, 0)),
            pl.BlockSpec((NNUM, D), lambda i: (0, 0)),
            pl.BlockSpec((NNUM, D), lambda i: (0, 0)),
        ],
        out_specs=pl.BlockSpec((bb, NNUM, D), lambda i: (i, 0, 0)),
        out_shape=jax.ShapeDtypeStruct((B, NNUM, D), jnp.float32),
    )(numerical, num_W, num_bias)


def _sget(ref, i, lane):
    kv = ref[pl.ds((i // 16) * 16, 16)]
    return jnp.max(jnp.where(lane == i % 16, kv, jnp.int32(-2147483648)))


def _sc_body(keys_hbm, offs_hbm, tabT_hbm, tail_hbm, bias_hbm, out_hbm,
             win_v, stag_v, tailbuf_v, oidx_v, tidx_v, bias_v,
             keys_v, offs_v, gsem):
    wid = lax.axis_index("s") * NC + lax.axis_index("c")
    u0 = wid * WPW
    u1 = jnp.minimum(u0 + WPW, WTOT)
    f_lo = u0 // NWINF
    f_hi = (u1 - 1) // NWINF + 1
    lane = lax.iota(jnp.int32, 16)
    lane0 = lane == 0
    pad_row = B * NCAT + (wid % 8)

    pltpu.sync_copy(bias_hbm, bias_v)

    def stage_entry(tot, rowvals, rowid):
        p = tot % 128
        pv = jnp.full((16,), p, jnp.int32)
        for k in range(D // 16):
            stag_v[p, pl.ds(16 * k, 16)] = rowvals[k]
        plsc.store_scatter(oidx_v.at[0], [pv],
                           jnp.full((16,), rowid, jnp.int32), mask=lane0)

        @pl.when(p == 127)
        def _():
            pltpu.sync_copy(stag_v, out_hbm.at[oidx_v.at[0]])

        return tot + 1

    def field_step(f, tot_in):
        base = f * NWINF
        ws = jnp.maximum(u0, base) - base
        we = jnp.minimum(u1, base + NWINF) - base
        pltpu.sync_copy(keys_hbm.at[f], keys_v)
        pltpu.sync_copy(offs_hbm.at[f], offs_v)
        bias_regs = [bias_v[f, pl.ds(16 * k, 16)] for k in range(D // 16)]

        def win_step(w, tot):
            ptr = _sget(offs_v, w, lane)
            end = _sget(offs_v, w + 1, lane)

            def stream_mode(tot0):
                w0 = w * WIN
                pltpu.sync_copy(
                    tabT_hbm.at[f, :, pl.ds(pl.multiple_of(w0, 128), WIN)],
                    win_v)

                def key_step(j, tot_k):
                    key = _sget(keys_v, j, lane)
                    v = key >> 12
                    pos = key & 4095
                    colv = jnp.full((16,), v - w0, jnp.int32)
                    rows = [plsc.load_gather(win_v, [lane + 16 * k, colv])
                            + bias_regs[k] for k in range(D // 16)]
                    return stage_entry(tot_k, rows, pos * NCAT + f)

                return lax.fori_loop(ptr, end, key_step, tot0)

            def tail_mode(tot0):
                for q in range(8):
                    plsc.store_scatter(
                        tidx_v.at[0], [lane + 16 * q],
                        jnp.full((16,), wid * 4, jnp.int32))

                def fill_step(j, c):
                    key = _sget(keys_v, j, lane)
                    trow = (key >> 12) - TAIL0 + f * TAILN
                    plsc.store_scatter(tidx_v.at[0],
                                       [jnp.full((16,), j - ptr, jnp.int32)],
                                       jnp.full((16,), trow, jnp.int32),
                                       mask=lane0)
                    return c

                lax.fori_loop(ptr, end, fill_step, 0)
                pltpu.async_copy(tail_hbm.at[tidx_v.at[0]], tailbuf_v,
                                 gsem).wait()

                def key_step(j, tot_k):
                    key = _sget(keys_v, j, lane)
                    pos = key & 4095
                    jl = j - ptr
                    rows = [tailbuf_v[jl, pl.ds(16 * k, 16)] + bias_regs[k]
                            for k in range(D // 16)]
                    return stage_entry(tot_k, rows, pos * NCAT + f)

                return lax.fori_loop(ptr, end, key_step, tot0)

            return lax.cond(w == NWINF - 1, tail_mode, stream_mode, tot)

        return lax.fori_loop(ws, we, win_step, tot_in)

    tot = lax.fori_loop(f_lo, f_hi, field_step, jnp.int32(0))

    @pl.when(tot % 128 != 0)
    def _():
        pfin = jnp.full((16,), tot % 128, jnp.int32)
        for q in range(8):
            lq = lane + 16 * q
            plsc.store_scatter(oidx_v.at[0], [lq],
                               jnp.full((16,), pad_row, jnp.int32),
                               mask=lq >= pfin)
        pltpu.sync_copy(stag_v, out_hbm.at[oidx_v.at[0]])


def _cat_embed(keys, offs, tablesT, tail_pad, bias_pad):
    mesh = plsc.VectorSubcoreMesh(core_axis_name="c", subcore_axis_name="s",
                                  num_cores=NC, num_subcores=NS)
    run = pl.kernel(
        _sc_body,
        out_type=jax.ShapeDtypeStruct((OUTROWS, 128), jnp.float32),
        mesh=mesh,
        scratch_types=[
            pltpu.VMEM((D, WIN), jnp.float32),
            pltpu.VMEM((128, 128), jnp.float32),
            pltpu.VMEM((128, 128), jnp.float32),
            pltpu.VMEM((1, 128), jnp.int32),
            pltpu.VMEM((1, 128), jnp.int32),
            pltpu.VMEM((NCAT, 128), jnp.float32),
            pltpu.VMEM((B,), jnp.int32),
            pltpu.VMEM((512,), jnp.int32),
            pltpu.SemaphoreType.DMA,
        ],
        compiler_params=pltpu.CompilerParams(use_tc_tiling_on_sc=True,
                                             needs_layout_passes=False),
    )
    return run(keys, offs, tablesT, tail_pad, bias_pad)


@jax.jit
def kernel(numerical, categorical, num_W, num_b, tables, num_token, cat_token,
           pos_enc):
    tablesT = jnp.transpose(tables, (0, 2, 1))
    catT = categorical.T
    keys = jnp.sort(catT * 4096 + jnp.arange(B, dtype=jnp.int32)[None, :],
                    axis=1)
    bounds = (jnp.arange(512, dtype=jnp.int32) << 20)
    offs = jnp.sum(keys[:, None, :] < bounds[None, :, None], axis=-1,
                   dtype=jnp.int32)
    tail_pad = jnp.pad(tables[:, TAIL0:, :],
                       ((0, 0), (0, 0), (0, 128 - D))).reshape(NCAT * TAILN,
                                                               128)
    bias_pad = jnp.pad(cat_token + pos_enc[NNUM:], ((0, 0), (0, 128 - D)))
    num_bias = num_b + num_token + pos_enc[:NNUM]
    num_e = _num_embed(numerical, num_W, num_bias)
    out_full = _cat_embed(keys, offs, tablesT, tail_pad, bias_pad)
    cat_e = out_full[:B * NCAT].reshape(B, NCAT, 128)[:, :, :D]
    return jnp.concatenate([num_e, cat_e], axis=1)


# speedup vs baseline: 2.6805x; 1.5776x over previous
import jax
import jax.numpy as jnp
from jax import lax
from jax.experimental import pallas as pl
from jax.experimental.pallas import tpu as pltpu
from jax.experimental.pallas import tpu_sc as plsc

B = 4096
NNUM = 13
NCAT = 26
V = 100000
D = 64

NC = 2
NS = 16
NW = NC * NS

WIN = 256
NWINF = 391
WTOT = NCAT * NWINF
WPW = (WTOT + NW - 1) // NW
TAIL0 = 390 * WIN
TAILN = V - TAIL0
OUTROWS = B * NCAT + 8
NBUF = 3


def _num_body(num_ref, w_ref, b_ref, out_ref):
    out_ref[...] = (num_ref[...][:, :, None] * w_ref[...][None, :, :]
                    + b_ref[...][None, :, :])


def _num_embed(numerical, num_W, num_bias):
    bb = 1024
    return pl.pallas_call(
        _num_body,
        grid=(B // bb,),
        in_specs=[
            pl.BlockSpec((bb, NNUM), lambda i: (i, 0)),
            pl.BlockSpec((NNUM, D), lambda i: (0, 0)),
            pl.BlockSpec((NNUM, D), lambda i: (0, 0)),
        ],
        out_specs=pl.BlockSpec((bb, NNUM, D), lambda i: (i, 0, 0)),
        out_shape=jax.ShapeDtypeStruct((B, NNUM, D), jnp.float32),
    )(numerical, num_W, num_bias)


def _sget(ref, i, lane):
    kv = ref[pl.ds((i // 16) * 16, 16)]
    return jnp.max(jnp.where(lane == i % 16, kv, jnp.int32(-2147483648)))


def _sc_body(keys_hbm, offs_hbm, tabT_hbm, tail_hbm, bias_hbm, out_hbm,
             win_v, stag_v, tailbuf_v, oidx_v, tidx_v, bias_v,
             keys_v, offs_v, gsems, tsem):
    wid = lax.axis_index("s") * NC + lax.axis_index("c")
    u0 = wid * WPW
    u1 = jnp.minimum(u0 + WPW, WTOT)
    f_lo = u0 // NWINF
    f_hi = (u1 - 1) // NWINF + 1
    lane = lax.iota(jnp.int32, 16)
    lane0 = lane == 0
    pad_row = B * NCAT + (wid % 8)

    pltpu.sync_copy(bias_hbm, bias_v)

    def stage_entry(tot, rowvals, rowid):
        p = tot % 128
        pv = jnp.full((16,), p, jnp.int32)
        for k in range(D // 16):
            stag_v[p, pl.ds(16 * k, 16)] = rowvals[k]
        plsc.store_scatter(oidx_v.at[0], [pv], rowid, mask=lane0)

        @pl.when(p == 127)
        def _():
            pltpu.sync_copy(stag_v, out_hbm.at[oidx_v.at[0]])

        return tot + 1

    def field_step(f, tot_in):
        base = f * NWINF
        ws = jnp.maximum(u0, base) - base
        we = jnp.minimum(u1, base + NWINF) - base
        pltpu.sync_copy(keys_hbm.at[f], keys_v)
        pltpu.sync_copy(offs_hbm.at[f], offs_v)
        bias_regs = [bias_v[f, pl.ds(16 * k, 16)] for k in range(D // 16)]
        nstream = jnp.minimum(we, NWINF - 1) - ws

        def fire(w, slot):
            pltpu.async_copy(
                tabT_hbm.at[f, :, pl.ds(pl.multiple_of(w * WIN, 128), WIN)],
                win_v.at[slot], gsems.at[slot])

        for q in range(NBUF):
            @pl.when(q < nstream)
            def _():
                fire(ws + q, q)

        def win_step(w, tot):
            slot = lax.rem(w - ws, NBUF)
            ptr = _sget(offs_v, w, lane)
            end = _sget(offs_v, w + 1, lane)
            w0 = w * WIN
            pltpu.make_async_copy(tabT_hbm.at[0, :, pl.ds(0, WIN)],
                                  win_v.at[slot], gsems.at[slot]).wait()

            def key_step(j, tot_k):
                key = plsc.load_gather(keys_v, [jnp.full((16,), j, jnp.int32)])
                colv = (key >> 12) - w0
                rowid = (key & 4095) * NCAT + f
                rows = [plsc.load_gather(win_v.at[slot],
                                         [lane + 16 * k, colv])
                        + bias_regs[k] for k in range(D // 16)]
                return stage_entry(tot_k, rows, rowid)

            tot2 = lax.fori_loop(ptr, end, key_step, tot)

            @pl.when(w + NBUF < ws + nstream)
            def _():
                fire(w + NBUF, slot)

            return tot2

        tot_mid = lax.fori_loop(ws, ws + nstream, win_step, tot_in)

        def tail_mode(tot0):
            ptr = _sget(offs_v, NWINF - 1, lane)
            end = _sget(offs_v, NWINF, lane)
            for q in range(8):
                plsc.store_scatter(tidx_v.at[0], [lane + 16 * q],
                                   jnp.full((16,), wid * 4, jnp.int32))

            def fill_step(j, c):
                key = plsc.load_gather(keys_v, [jnp.full((16,), j, jnp.int32)])
                trow = (key >> 12) - TAIL0 + f * TAILN
                plsc.store_scatter(tidx_v.at[0],
                                   [jnp.full((16,), j - ptr, jnp.int32)],
                                   trow, mask=lane0)
                return c

            lax.fori_loop(ptr, end, fill_step, 0)
            pltpu.async_copy(tail_hbm.at[tidx_v.at[0]], tailbuf_v,
                             tsem).wait()

            def key_step(j, tot_k):
                key = plsc.load_gather(keys_v, [jnp.full((16,), j, jnp.int32)])
                rowid = (key & 4095) * NCAT + f
                jl = j - ptr
                rows = [tailbuf_v[jl, pl.ds(16 * k, 16)] + bias_regs[k]
                        for k in range(D // 16)]
                return stage_entry(tot_k, rows, rowid)

            return lax.fori_loop(ptr, end, key_step, tot0)

        return lax.cond(we == NWINF, tail_mode, lambda t: t, tot_mid)

    tot = lax.fori_loop(f_lo, f_hi, field_step, jnp.int32(0))

    @pl.when(tot % 128 != 0)
    def _():
        pfin = jnp.full((16,), tot % 128, jnp.int32)
        for q in range(8):
            lq = lane + 16 * q
            plsc.store_scatter(oidx_v.at[0], [lq],
                               jnp.full((16,), pad_row, jnp.int32),
                               mask=lq >= pfin)
        pltpu.sync_copy(stag_v, out_hbm.at[oidx_v.at[0]])


def _cat_embed(keys, offs, tablesT, tail_pad, bias_pad):
    mesh = plsc.VectorSubcoreMesh(core_axis_name="c", subcore_axis_name="s",
                                  num_cores=NC, num_subcores=NS)
    run = pl.kernel(
        _sc_body,
        out_type=jax.ShapeDtypeStruct((OUTROWS, 128), jnp.float32),
        mesh=mesh,
        scratch_types=[
            pltpu.VMEM((NBUF, D, WIN), jnp.float32),
            pltpu.VMEM((128, 128), jnp.float32),
            pltpu.VMEM((128, 128), jnp.float32),
            pltpu.VMEM((1, 128), jnp.int32),
            pltpu.VMEM((1, 128), jnp.int32),
            pltpu.VMEM((NCAT, 128), jnp.float32),
            pltpu.VMEM((B,), jnp.int32),
            pltpu.VMEM((512,), jnp.int32),
            pltpu.SemaphoreType.DMA((NBUF,)),
            pltpu.SemaphoreType.DMA,
        ],
        compiler_params=pltpu.CompilerParams(use_tc_tiling_on_sc=True,
                                             needs_layout_passes=False),
    )
    return run(keys, offs, tablesT, tail_pad, bias_pad)


@jax.jit
def kernel(numerical, categorical, num_W, num_b, tables, num_token, cat_token,
           pos_enc):
    tablesT = jnp.transpose(tables, (0, 2, 1))
    catT = categorical.T
    keys = jnp.sort(catT * 4096 + jnp.arange(B, dtype=jnp.int32)[None, :],
                    axis=1)
    bounds = (jnp.arange(512, dtype=jnp.int32) << 20)
    offs = jnp.sum(keys[:, None, :] < bounds[None, :, None], axis=-1,
                   dtype=jnp.int32)
    tail_pad = jnp.pad(tables[:, TAIL0:, :],
                       ((0, 0), (0, 0), (0, 128 - D))).reshape(NCAT * TAILN,
                                                               128)
    bias_pad = jnp.pad(cat_token + pos_enc[NNUM:], ((0, 0), (0, 128 - D)))
    num_bias = num_b + num_token + pos_enc[:NNUM]
    num_e = _num_embed(numerical, num_W, num_bias)
    out_full = _cat_embed(keys, offs, tablesT, tail_pad, bias_pad)
    cat_e = out_full[:B * NCAT].reshape(B, NCAT, 128)[:, :, :D]
    return jnp.concatenate([num_e, cat_e], axis=1)


# speedup vs baseline: 2.8620x; 1.0677x over previous
import jax
import jax.numpy as jnp
from jax import lax
from jax.experimental import pallas as pl
from jax.experimental.pallas import tpu as pltpu
from jax.experimental.pallas import tpu_sc as plsc

B = 4096
NNUM = 13
NCAT = 26
V = 100000
D = 64

NC = 2
NS = 16
NW = NC * NS

WIN = 256
NWINF = 391
WTOT = NCAT * NWINF
WPW = (WTOT + NW - 1) // NW
TAIL0 = 390 * WIN
TAILN = V - TAIL0
OUTROWS = B * NCAT + 8
NBUF = 4


def _num_body(num_ref, w_ref, b_ref, out_ref):
    out_ref[...] = (num_ref[...][:, :, None] * w_ref[...][None, :, :]
                    + b_ref[...][None, :, :])


def _num_embed(numerical, num_W, num_bias):
    bb = 1024
    return pl.pallas_call(
        _num_body,
        grid=(B // bb,),
        in_specs=[
            pl.BlockSpec((bb, NNUM), lambda i: (i, 0)),
            pl.BlockSpec((NNUM, D), lambda i: (0, 0)),
            pl.BlockSpec((NNUM, D), lambda i: (0, 0)),
        ],
        out_specs=pl.BlockSpec((bb, NNUM, D), lambda i: (i, 0, 0)),
        out_shape=jax.ShapeDtypeStruct((B, NNUM, D), jnp.float32),
    )(numerical, num_W, num_bias)


def _sget(ref, i, lane):
    kv = ref[pl.ds((i // 16) * 16, 16)]
    return jnp.max(jnp.where(lane == i % 16, kv, jnp.int32(-2147483648)))


def _sc_body(keys_hbm, offs_hbm, tabT_hbm, tail_hbm, bias_hbm, out_hbm,
             win_v, stag_v, tailbuf_v, oidx_v, tidx_v, bias_v,
             keys_v, offs_v, gsems, ssems, tsem):
    wid = lax.axis_index("s") * NC + lax.axis_index("c")
    u0 = wid * WPW
    u1 = jnp.minimum(u0 + WPW, WTOT)
    f_lo = u0 // NWINF
    f_hi = (u1 - 1) // NWINF + 1
    lane = lax.iota(jnp.int32, 16)
    lane0 = lane == 0
    pad_row = B * NCAT + (wid % 8)

    pltpu.sync_copy(bias_hbm, bias_v)

    def stage_entry(tot, rowvals, rowid):
        p = tot % 128
        blk = tot // 128
        sl = lax.rem(blk, 2)
        pv = jnp.full((16,), p, jnp.int32)

        @pl.when((p == 0) & (blk >= 2))
        def _():
            pltpu.make_async_copy(tabT_hbm.at[0, :, pl.ds(0, WIN)],
                                  stag_v.at[sl], ssems.at[sl]).wait()

        for k in range(D // 16):
            stag_v[sl, p, pl.ds(16 * k, 16)] = rowvals[k]
        plsc.store_scatter(oidx_v.at[sl], [pv], rowid, mask=lane0)

        @pl.when(p == 127)
        def _():
            pltpu.async_copy(stag_v.at[sl], out_hbm.at[oidx_v.at[sl]],
                             ssems.at[sl])

        return tot + 1

    def field_step(f, tot_in):
        base = f * NWINF
        ws = jnp.maximum(u0, base) - base
        we = jnp.minimum(u1, base + NWINF) - base
        pltpu.sync_copy(keys_hbm.at[f], keys_v)
        pltpu.sync_copy(offs_hbm.at[f], offs_v)
        bias_regs = [bias_v[f, pl.ds(16 * k, 16)] for k in range(D // 16)]
        nstream = jnp.minimum(we, NWINF - 1) - ws

        def fire(w, slot):
            pltpu.async_copy(
                tabT_hbm.at[f, :, pl.ds(pl.multiple_of(w * WIN, 128), WIN)],
                win_v.at[slot], gsems.at[slot])

        for q in range(NBUF):
            @pl.when(q < nstream)
            def _():
                fire(ws + q, q)

        def win_step(w, tot):
            slot = lax.rem(w - ws, NBUF)
            ptr = _sget(offs_v, w, lane)
            end = _sget(offs_v, w + 1, lane)
            w0 = w * WIN
            pltpu.make_async_copy(tabT_hbm.at[0, :, pl.ds(0, WIN)],
                                  win_v.at[slot], gsems.at[slot]).wait()

            def key_step(j, tot_k):
                key = plsc.load_gather(keys_v, [jnp.full((16,), j, jnp.int32)])
                colv = (key >> 12) - w0
                rowid = (key & 4095) * NCAT + f
                rows = [plsc.load_gather(win_v.at[slot],
                                         [lane + 16 * k, colv])
                        + bias_regs[k] for k in range(D // 16)]
                return stage_entry(tot_k, rows, rowid)

            tot2 = lax.fori_loop(ptr, end, key_step, tot)

            @pl.when(w + NBUF < ws + nstream)
            def _():
                fire(w + NBUF, slot)

            return tot2

        tot_mid = lax.fori_loop(ws, ws + nstream, win_step, tot_in)

        def tail_mode(tot0):
            ptr = _sget(offs_v, NWINF - 1, lane)
            end = _sget(offs_v, NWINF, lane)
            for q in range(8):
                plsc.store_scatter(tidx_v.at[0], [lane + 16 * q],
                                   jnp.full((16,), wid * 4, jnp.int32))

            def fill_step(j, c):
                key = plsc.load_gather(keys_v, [jnp.full((16,), j, jnp.int32)])
                trow = (key >> 12) - TAIL0 + f * TAILN
                plsc.store_scatter(tidx_v.at[0],
                                   [jnp.full((16,), j - ptr, jnp.int32)],
                                   trow, mask=lane0)
                return c

            lax.fori_loop(ptr, end, fill_step, 0)
            pltpu.async_copy(tail_hbm.at[tidx_v.at[0]], tailbuf_v,
                             tsem).wait()

            def key_step(j, tot_k):
                key = plsc.load_gather(keys_v, [jnp.full((16,), j, jnp.int32)])
                rowid = (key & 4095) * NCAT + f
                jl = j - ptr
                rows = [tailbuf_v[jl, pl.ds(16 * k, 16)] + bias_regs[k]
                        for k in range(D // 16)]
                return stage_entry(tot_k, rows, rowid)

            return lax.fori_loop(ptr, end, key_step, tot0)

        return lax.cond(we == NWINF, tail_mode, lambda t: t, tot_mid)

    tot = lax.fori_loop(f_lo, f_hi, field_step, jnp.int32(0))

    blk_f = tot // 128

    @pl.when(blk_f >= 1)
    def _():
        pltpu.make_async_copy(tabT_hbm.at[0, :, pl.ds(0, WIN)],
                              stag_v.at[lax.rem(blk_f - 1, 2)],
                              ssems.at[lax.rem(blk_f - 1, 2)]).wait()

    @pl.when(tot % 128 != 0)
    def _():
        slf = lax.rem(blk_f, 2)
        pfin = jnp.full((16,), tot % 128, jnp.int32)
        for q in range(8):
            lq = lane + 16 * q
            plsc.store_scatter(oidx_v.at[slf], [lq],
                               jnp.full((16,), pad_row, jnp.int32),
                               mask=lq >= pfin)
        pltpu.async_copy(stag_v.at[slf], out_hbm.at[oidx_v.at[slf]],
                         ssems.at[slf]).wait()


def _cat_embed(keys, offs, tablesT, tail_pad, bias_pad):
    mesh = plsc.VectorSubcoreMesh(core_axis_name="c", subcore_axis_name="s",
                                  num_cores=NC, num_subcores=NS)
    run = pl.kernel(
        _sc_body,
        out_type=jax.ShapeDtypeStruct((OUTROWS, 128), jnp.float32),
        mesh=mesh,
        scratch_types=[
            pltpu.VMEM((NBUF, D, WIN), jnp.float32),
            pltpu.VMEM((2, 128, 128), jnp.float32),
            pltpu.VMEM((128, 128), jnp.float32),
            pltpu.VMEM((2, 128), jnp.int32),
            pltpu.VMEM((1, 128), jnp.int32),
            pltpu.VMEM((NCAT, 128), jnp.float32),
            pltpu.VMEM((B,), jnp.int32),
            pltpu.VMEM((512,), jnp.int32),
            pltpu.SemaphoreType.DMA((NBUF,)),
            pltpu.SemaphoreType.DMA((2,)),
            pltpu.SemaphoreType.DMA,
        ],
        compiler_params=pltpu.CompilerParams(use_tc_tiling_on_sc=True,
                                             needs_layout_passes=False),
    )
    return run(keys, offs, tablesT, tail_pad, bias_pad)


@jax.jit
def kernel(numerical, categorical, num_W, num_b, tables, num_token, cat_token,
           pos_enc):
    tablesT = jnp.transpose(tables, (0, 2, 1))
    catT = categorical.T
    keys = jnp.sort(catT * 4096 + jnp.arange(B, dtype=jnp.int32)[None, :],
                    axis=1)
    bounds = (jnp.arange(512, dtype=jnp.int32) << 20)
    offs = jnp.sum(keys[:, None, :] < bounds[None, :, None], axis=-1,
                   dtype=jnp.int32)
    tail_pad = jnp.pad(tables[:, TAIL0:, :],
                       ((0, 0), (0, 0), (0, 128 - D))).reshape(NCAT * TAILN,
                                                               128)
    bias_pad = jnp.pad(cat_token + pos_enc[NNUM:], ((0, 0), (0, 128 - D)))
    num_bias = num_b + num_token + pos_enc[:NNUM]
    num_e = _num_embed(numerical, num_W, num_bias)
    out_full = _cat_embed(keys, offs, tablesT, tail_pad, bias_pad)
    cat_e = out_full[:B * NCAT].reshape(B, NCAT, 128)[:, :, :D]
    return jnp.concatenate([num_e, cat_e], axis=1)
